# baseline (speedup 1.0000x reference)
import functools

import numpy as np

import jax
import jax.numpy as jnp
from jax import lax
from jax.experimental import pallas as pl
from jax.experimental.pallas import tpu as pltpu
from jax.experimental.pallas import tpu_sc as plsc

_F, _B, _L, _ZCH, _D = 26, 4096, 20, 100000, 64
_NW = 32
_BAGS_W_F = _B // _NW
_CB = 32
_NCHUNK = _F * (_BAGS_W_F // _CB)
_IDS_ROWS = _CB * _L // 128
_VROWS = _F * _B * _L // 128


def _remap_vec(v_i32, foff_i32):
    _M = jnp.uint32(_ZCH)
    v = v_i32.astype(jnp.uint32)
    h = v * jnp.uint32(2654435761)
    h = h ^ (h >> jnp.uint32(16))
    u = h >> jnp.uint32(16)
    w = h & jnp.uint32(0xFFFF)
    t = (u * jnp.uint32(256)) % _M
    t = (t * jnp.uint32(256)) % _M
    idx = (t + w) % _M
    return idx.astype(jnp.int32) + foff_i32


@functools.partial(
    pl.kernel,
    mesh=plsc.VectorSubcoreMesh(core_axis_name="c", subcore_axis_name="s"),
    out_type=jax.ShapeDtypeStruct((_F * _B, _D), jnp.float32),
    scratch_types=[
        pltpu.VMEM((4 * _IDS_ROWS, 128), jnp.int32),
        pltpu.VMEM((2, _CB * _L, _D // 2), jnp.float32),

        pltpu.VMEM((2, _CB, _D), jnp.float32),
        pltpu.SemaphoreType.DMA,
        pltpu.SemaphoreType.DMA,
        pltpu.SemaphoreType.DMA,
        pltpu.SemaphoreType.DMA,
        pltpu.SemaphoreType.DMA,
        pltpu.SemaphoreType.DMA,
        pltpu.SemaphoreType.DMA,
        pltpu.SemaphoreType.DMA,
    ],
    compiler_params=pltpu.CompilerParams(
        use_tc_tiling_on_sc=False, needs_layout_passes=False
    ),
)
def _sc_embedding_bag(
    vals_hbm, tables_hbm, out_hbm,
    ids_v, rows_v, out_v,
    si0, si1, si2, si3, sg0, sg1, so0, so1,
):
    sem_i = (si0, si1, si2, si3)
    sem_g = (sg0, sg1)
    sem_o = (so0, so1)
    wid = lax.axis_index("s") * 2 + lax.axis_index("c")

    def bag0_of(t):
        f = lax.shift_right_logical(t, 2)
        c = t & 3
        return f * _B + wid * _BAGS_W_F + c * _CB

    def idrow0_of(t):
        return bag0_of(t) * _L // 128

    def foff_of(t):
        return lax.shift_right_logical(t, 2) * jnp.int32(_ZCH)

    def ids_block(slot):
        return ids_v.at[pl.ds(slot * _IDS_ROWS, _IDS_ROWS)]

    def fire_ids(t, slot):
        pltpu.async_copy(
            vals_hbm.at[pl.ds(idrow0_of(t), _IDS_ROWS)], ids_block(slot),
            sem_i[slot],
        )

    def wait_ids(t, slot):
        pltpu.make_async_copy(
            vals_hbm.at[pl.ds(idrow0_of(t), _IDS_ROWS)], ids_block(slot),
            sem_i[slot],
        ).wait()

    def hash_slot(t, slot):
        foff = foff_of(t)
        for j in range(_IDS_ROWS):
            r = slot * _IDS_ROWS + j
            for k in range(8):
                sl = pl.ds(k * 16, 16)
                ids_v[r, sl] = _remap_vec(ids_v[r, sl], foff)

    def fire_gathers(slot, p):
        for j in range(_IDS_ROWS):
            pltpu.async_copy(
                tables_hbm.at[ids_v.at[slot * _IDS_ROWS + j]],
                rows_v.at[p, pl.ds(j * 128, 128)],
                sem_g[p],
            )

    def wait_gathers(slot, p):
        for j in range(_IDS_ROWS):
            pltpu.make_async_copy(
                tables_hbm.at[ids_v.at[slot * _IDS_ROWS + j]],
                rows_v.at[p, pl.ds(j * 128, 128)],
                sem_g[p],
            ).wait()

    def pool(p):
        def bag_body(b, _):
            r0 = b * _L
            accs = [None] * 4
            for l in range(_L):
                for k in range(2):
                    v16 = rows_v[p, r0 + l, pl.ds(k * 16, 16)]
                    v32 = plsc.bitcast(v16, jnp.bfloat16)
                    ev, od = plsc.unpack(v32, format=plsc.PackFormat.INTERLEAVED)
                    if l == 0:
                        accs[2 * k], accs[2 * k + 1] = ev, od
                    else:
                        accs[2 * k] = accs[2 * k] + ev
                        accs[2 * k + 1] = accs[2 * k + 1] + od
            for q in range(4):
                out_v[p, b, pl.ds(q * 16, 16)] = accs[q]
            return 0

        lax.fori_loop(0, _CB, bag_body, 0)

    def fire_out(t, p):
        pltpu.async_copy(
            out_v.at[p], out_hbm.at[pl.ds(bag0_of(t), _CB)], sem_o[p]
        )

    def wait_out(p):
        pltpu.make_async_copy(
            out_v.at[p], out_hbm.at[pl.ds(0, _CB)], sem_o[p]
        ).wait()

    t0 = jnp.int32(0)
    fire_ids(t0, 0)
    wait_ids(t0, 0)
    hash_slot(t0, 0)
    fire_gathers(0, 0)
    fire_ids(t0 + 1, 1)

    def outer(i, _):
        for j in range(4):
            t = i * 4 + j
            sj2, sj1 = (j + 2) % 4, (j + 1) % 4
            p, p1 = j % 2, (j + 1) % 2

            @pl.when(t + 2 < _NCHUNK)
            def _():
                fire_ids(t + 2, sj2)

            @pl.when(t + 1 < _NCHUNK)
            def _():
                wait_ids(t + 1, sj1)
                hash_slot(t + 1, sj1)
                fire_gathers(sj1, p1)

            wait_gathers(j % 4, p)

            @pl.when(t >= 2)
            def _():
                wait_out(p)

            pool(p)
            fire_out(t, p)
        return 0

    lax.fori_loop(0, _NCHUNK // 4, outer, 0)
    wait_out(0)
    wait_out(1)


_COL_INV = np.zeros(_D, dtype=np.int32)
for _c in range(_D):
    _COL_INV[_c] = 32 * (_c // 32) + 16 * (_c % 2) + (_c % 32) // 2


def kernel(values, tables):
    vals = values.astype(jnp.int32).reshape(_VROWS, 128)
    tabs = lax.bitcast_convert_type(
        tables.reshape(_F * _ZCH, _D)
        .astype(jnp.bfloat16)
        .reshape(_F * _ZCH, _D // 2, 2),
        jnp.float32,
    )
    out = _sc_embedding_bag(vals, tabs)
    return out.reshape(_F, _B, _D)[:, :, _COL_INV]


# speedup vs baseline: 2.3789x; 2.3789x over previous
import functools

import numpy as np

import jax
import jax.numpy as jnp
from jax import lax
from jax.experimental import pallas as pl
from jax.experimental.pallas import tpu as pltpu
from jax.experimental.pallas import tpu_sc as plsc

_F, _B, _L, _ZCH, _D = 26, 4096, 20, 100000, 64
_NW = 32
_BAGS_W_F = _B // _NW
_CB = 32
_NCHUNK = _F * (_BAGS_W_F // _CB)
_IDS_ROWS = _CB * _L // 128
_VROWS = _F * _B * _L // 128


_TCR = 2080
_TCQ = _TCR // 4


def _remap_vec(v_i32, foff_u32):
    _M = jnp.uint32(_ZCH)
    v = v_i32.astype(jnp.uint32)
    h = v * jnp.uint32(2654435761)
    h = h ^ (h >> jnp.uint32(16))
    u = h >> jnp.uint32(16)
    w = h & jnp.uint32(0xFFFF)
    t = (u * jnp.uint32(256)) % _M
    t = (t * jnp.uint32(256)) % _M
    r = (t + w) % _M + foff_u32
    g = r // jnp.uint32(_TCR)
    lr = r - g * jnp.uint32(_TCR)
    k = lr // jnp.uint32(_TCQ)
    j2 = lr - k * jnp.uint32(_TCQ)
    m = g * jnp.uint32(_TCR) + j2 * jnp.uint32(4) + k
    return m.astype(jnp.int32)


def _tc_pack_body(x_ref, o_ref):
    x = x_ref[...]
    u = lax.bitcast_convert_type(x, jnp.uint32)
    rnd = (u + jnp.uint32(0x7FFF) + ((u >> jnp.uint32(16)) & jnp.uint32(1))) \
        >> jnp.uint32(16)
    ev = rnd[:, : _D // 2]
    od = rnd[:, _D // 2:]
    p = ev | (od << jnp.uint32(16))
    f = lax.bitcast_convert_type(p, jnp.float32)
    o_ref[...] = jnp.concatenate(
        [f[k * _TCQ:(k + 1) * _TCQ, :] for k in range(4)], axis=1
    )


_tc_pack = pl.pallas_call(
    _tc_pack_body,
    grid=(_F * _ZCH // _TCR,),
    in_specs=[pl.BlockSpec((_TCR, _D), lambda i: (i, 0))],
    out_specs=pl.BlockSpec((_TCQ, 2 * _D), lambda i: (i, 0)),
    out_shape=jax.ShapeDtypeStruct((_F * _ZCH // 4, 2 * _D), jnp.float32),
)


@functools.partial(
    pl.kernel,
    mesh=plsc.VectorSubcoreMesh(core_axis_name="c", subcore_axis_name="s"),
    out_type=jax.ShapeDtypeStruct((_F * _B, _D), jnp.float32),
    scratch_types=[
        pltpu.VMEM((4 * _IDS_ROWS, 128), jnp.int32),
        pltpu.VMEM((2, _CB * _L, _D // 2), jnp.float32),

        pltpu.VMEM((2, _CB, _D), jnp.float32),
        pltpu.SemaphoreType.DMA,
        pltpu.SemaphoreType.DMA,
        pltpu.SemaphoreType.DMA,
        pltpu.SemaphoreType.DMA,
        pltpu.SemaphoreType.DMA,
        pltpu.SemaphoreType.DMA,
        pltpu.SemaphoreType.DMA,
        pltpu.SemaphoreType.DMA,
    ],
    compiler_params=pltpu.CompilerParams(
        use_tc_tiling_on_sc=False, needs_layout_passes=False
    ),
)
def _sc_embedding_bag(
    vals_hbm, tables_hbm, out_hbm,
    ids_v, rows_v, out_v,
    si0, si1, si2, si3, sg0, sg1, so0, so1,
):
    sem_i = (si0, si1, si2, si3)
    sem_g = (sg0, sg1)
    sem_o = (so0, so1)
    wid = lax.axis_index("s") * 2 + lax.axis_index("c")

    def bag0_of(t):
        f = lax.shift_right_logical(t, 2)
        c = t & 3
        return f * _B + wid * _BAGS_W_F + c * _CB

    def idrow0_of(t):
        return bag0_of(t) * _L // 128

    def foff_of(t):
        return lax.shift_right_logical(t, 2).astype(jnp.uint32) * jnp.uint32(_ZCH)

    def ids_block(slot):
        return ids_v.at[pl.ds(slot * _IDS_ROWS, _IDS_ROWS)]

    def fire_ids(t, slot):
        pltpu.async_copy(
            vals_hbm.at[pl.ds(idrow0_of(t), _IDS_ROWS)], ids_block(slot),
            sem_i[slot],
        )

    def wait_ids(t, slot):
        pltpu.make_async_copy(
            vals_hbm.at[pl.ds(idrow0_of(t), _IDS_ROWS)], ids_block(slot),
            sem_i[slot],
        ).wait()

    def hash_slot(t, slot):
        foff = foff_of(t)
        for j in range(_IDS_ROWS):
            r = slot * _IDS_ROWS + j
            for k in range(8):
                sl = pl.ds(k * 16, 16)
                ids_v[r, sl] = _remap_vec(ids_v[r, sl], foff)

    def fire_gathers(slot, p):
        for j in range(_IDS_ROWS):
            pltpu.async_copy(
                tables_hbm.at[ids_v.at[slot * _IDS_ROWS + j]],
                rows_v.at[p, pl.ds(j * 128, 128)],
                sem_g[p],
            )

    def wait_gathers(slot, p):
        for j in range(_IDS_ROWS):
            pltpu.make_async_copy(
                tables_hbm.at[ids_v.at[slot * _IDS_ROWS + j]],
                rows_v.at[p, pl.ds(j * 128, 128)],
                sem_g[p],
            ).wait()

    def pool(p):
        def bag_body(b, _):
            r0 = b * _L
            accs = [None] * 4
            for l in range(_L):
                for k in range(2):
                    v16 = rows_v[p, r0 + l, pl.ds(k * 16, 16)]
                    v32 = plsc.bitcast(v16, jnp.bfloat16)
                    ev, od = plsc.unpack(v32, format=plsc.PackFormat.INTERLEAVED)
                    if l == 0:
                        accs[2 * k], accs[2 * k + 1] = ev, od
                    else:
                        accs[2 * k] = accs[2 * k] + ev
                        accs[2 * k + 1] = accs[2 * k + 1] + od
            for q in range(4):
                out_v[p, b, pl.ds(q * 16, 16)] = accs[q]
            return 0

        lax.fori_loop(0, _CB, bag_body, 0)

    def fire_out(t, p):
        pltpu.async_copy(
            out_v.at[p], out_hbm.at[pl.ds(bag0_of(t), _CB)], sem_o[p]
        )

    def wait_out(p):
        pltpu.make_async_copy(
            out_v.at[p], out_hbm.at[pl.ds(0, _CB)], sem_o[p]
        ).wait()

    t0 = jnp.int32(0)
    fire_ids(t0, 0)
    wait_ids(t0, 0)
    hash_slot(t0, 0)
    fire_gathers(0, 0)
    fire_ids(t0 + 1, 1)

    def outer(i, _):
        for j in range(4):
            t = i * 4 + j
            sj2, sj1 = (j + 2) % 4, (j + 1) % 4
            p, p1 = j % 2, (j + 1) % 2

            @pl.when(t + 2 < _NCHUNK)
            def _():
                fire_ids(t + 2, sj2)

            @pl.when(t + 1 < _NCHUNK)
            def _():
                wait_ids(t + 1, sj1)
                hash_slot(t + 1, sj1)
                fire_gathers(sj1, p1)

            wait_gathers(j % 4, p)

            @pl.when(t >= 2)
            def _():
                wait_out(p)

            pool(p)
            fire_out(t, p)
        return 0

    lax.fori_loop(0, _NCHUNK // 4, outer, 0)
    wait_out(0)
    wait_out(1)


_COL_INV = np.concatenate([
    np.arange(0, 16), np.arange(32, 48), np.arange(16, 32), np.arange(48, 64)
]).astype(np.int32)


def kernel(values, tables):
    vals = values.astype(jnp.int32).reshape(_VROWS, 128)
    tabs = _tc_pack(tables.reshape(_F * _ZCH, _D)).reshape(_F * _ZCH, _D // 2)
    out = _sc_embedding_bag(vals, tabs)
    return out.reshape(_F, _B, _D)[:, :, _COL_INV]


# speedup vs baseline: 2.4804x; 1.0427x over previous
import functools

import jax
import jax.numpy as jnp
from jax import lax
from jax.experimental import pallas as pl
from jax.experimental.pallas import tpu as pltpu
from jax.experimental.pallas import tpu_sc as plsc

_F, _B, _L, _ZCH, _D = 26, 4096, 20, 100000, 64
_NW = 32
_BAGS_W_F = _B // _NW
_CB = 32
_NCHUNK = _F * (_BAGS_W_F // _CB)
_IDS_ROWS = _CB * _L // 128
_VROWS = _F * _B * _L // 128


def _remap_vec(v_i32, foff_i32):
    _M = jnp.uint32(_ZCH)
    v = v_i32.astype(jnp.uint32)
    h = v * jnp.uint32(2654435761)
    h = h ^ (h >> jnp.uint32(16))
    u = h >> jnp.uint32(16)
    w = h & jnp.uint32(0xFFFF)
    t = (u * jnp.uint32(256)) % _M
    t = (t * jnp.uint32(256)) % _M
    idx = (t + w) % _M
    return idx.astype(jnp.int32) + foff_i32


@functools.partial(
    pl.kernel,
    mesh=plsc.VectorSubcoreMesh(core_axis_name="c", subcore_axis_name="s"),
    out_type=jax.ShapeDtypeStruct((_F * _B, _D), jnp.float32),
    scratch_types=[
        pltpu.VMEM((4 * _IDS_ROWS, 128), jnp.int32),
        pltpu.VMEM((2, _CB * _L, _D), jnp.float32),
        pltpu.VMEM((2, _CB, _D), jnp.float32),
        pltpu.SemaphoreType.DMA,
        pltpu.SemaphoreType.DMA,
        pltpu.SemaphoreType.DMA,
        pltpu.SemaphoreType.DMA,
        pltpu.SemaphoreType.DMA,
        pltpu.SemaphoreType.DMA,
        pltpu.SemaphoreType.DMA,
        pltpu.SemaphoreType.DMA,
    ],
    compiler_params=pltpu.CompilerParams(use_tc_tiling_on_sc=False),
)
def _sc_embedding_bag(
    vals_hbm, tables_hbm, out_hbm,
    ids_v, rows_v, out_v,
    si0, si1, si2, si3, sg0, sg1, so0, so1,
):
    sem_i = (si0, si1, si2, si3)
    sem_g = (sg0, sg1)
    sem_o = (so0, so1)
    wid = lax.axis_index("s") * 2 + lax.axis_index("c")

    def bag0_of(t):
        f = lax.shift_right_logical(t, 2)
        c = t & 3
        return f * _B + wid * _BAGS_W_F + c * _CB

    def idrow0_of(t):
        return bag0_of(t) * _L // 128

    def foff_of(t):
        return lax.shift_right_logical(t, 2) * jnp.int32(_ZCH)

    def ids_block(slot):
        return ids_v.at[pl.ds(slot * _IDS_ROWS, _IDS_ROWS)]

    def fire_ids(t, slot):
        pltpu.async_copy(
            vals_hbm.at[pl.ds(idrow0_of(t), _IDS_ROWS)], ids_block(slot),
            sem_i[slot],
        )

    def wait_ids(t, slot):
        pltpu.make_async_copy(
            vals_hbm.at[pl.ds(idrow0_of(t), _IDS_ROWS)], ids_block(slot),
            sem_i[slot],
        ).wait()

    def hash_slot(t, slot):
        foff = foff_of(t)
        for j in range(_IDS_ROWS):
            r = slot * _IDS_ROWS + j
            for k in range(8):
                sl = pl.ds(k * 16, 16)
                ids_v[r, sl] = _remap_vec(ids_v[r, sl], foff)

    def fire_gathers(slot, p):
        for j in range(_IDS_ROWS):
            pltpu.async_copy(
                tables_hbm.at[ids_v.at[slot * _IDS_ROWS + j]],
                rows_v.at[p, pl.ds(j * 128, 128)],
                sem_g[p],
            )

    def wait_gathers(slot, p):
        for j in range(_IDS_ROWS):
            pltpu.make_async_copy(
                tables_hbm.at[ids_v.at[slot * _IDS_ROWS + j]],
                rows_v.at[p, pl.ds(j * 128, 128)],
                sem_g[p],
            ).wait()

    def pool(p):
        def bag_body(b, _):
            r0 = b * _L
            for d4 in range(_D // 16):
                sl = pl.ds(d4 * 16, 16)
                acc = rows_v[p, r0, sl]
                for l in range(1, _L):
                    acc = acc + rows_v[p, r0 + l, sl]
                out_v[p, b, sl] = acc
            return 0

        lax.fori_loop(0, _CB, bag_body, 0)

    def fire_out(t, p):
        pltpu.async_copy(
            out_v.at[p], out_hbm.at[pl.ds(bag0_of(t), _CB)], sem_o[p]
        )

    def wait_out(p):
        pltpu.make_async_copy(
            out_v.at[p], out_hbm.at[pl.ds(0, _CB)], sem_o[p]
        ).wait()

    t0 = jnp.int32(0)
    fire_ids(t0, 0)
    wait_ids(t0, 0)
    hash_slot(t0, 0)
    fire_gathers(0, 0)
    fire_ids(t0 + 1, 1)

    def outer(i, _):
        for j in range(4):
            t = i * 4 + j
            sj2, sj1 = (j + 2) % 4, (j + 1) % 4
            p, p1 = j % 2, (j + 1) % 2

            @pl.when(t + 2 < _NCHUNK)
            def _():
                fire_ids(t + 2, sj2)

            @pl.when(t + 1 < _NCHUNK)
            def _():
                wait_ids(t + 1, sj1)
                hash_slot(t + 1, sj1)
                fire_gathers(sj1, p1)

            wait_gathers(j % 4, p)

            @pl.when(t >= 2)
            def _():
                wait_out(p)

            pool(p)
            fire_out(t, p)
        return 0

    lax.fori_loop(0, _NCHUNK // 4, outer, 0)
    wait_out(0)
    wait_out(1)


def kernel(values, tables):
    vals = values.astype(jnp.int32).reshape(_VROWS, 128)
    tabs = tables.reshape(_F * _ZCH, _D)
    out = _sc_embedding_bag(vals, tabs)
    return out.reshape(_F, _B, _D)
